# Initial kernel scaffold; baseline (speedup 1.0000x reference)
import jax
import jax.numpy as jnp
from jax.experimental import pallas as pl


def kernel(x, edge_index, Wl1, bl1, Wr1, bn_gamma, bn_beta, Wl2, bl2, Wr2):
    raise NotImplementedError("write your pallas kernel here")


# speedup vs baseline: 3.2102x; 3.2102x over previous
import functools

import jax
import jax.numpy as jnp
from jax import lax
from jax.experimental import pallas as pl
from jax.experimental.pallas import tpu as pltpu
from jax.experimental.pallas import tpu_sc as plsc

N = 10000
E = 320000
D = 128
HD = D // 2
NC = 2
NS = 16
L = 16
EPS = E // NS
CHUNK = 80
STEPS = EPS // CHUNK
NPAD = 10240
RPS = NPAD // NS


def _seg_sum_call(feats2, src, dst, zrow, zcnt, ones_pat, with_counts):
  mesh = plsc.VectorSubcoreMesh(
      core_axis_name="c", subcore_axis_name="s", num_cores=NC, num_subcores=NS)

  out_type = [jax.ShapeDtypeStruct((NC * NPAD, HD), jnp.float32)]
  scratch = {
      "sidx": pltpu.VMEM((CHUNK,), jnp.int32),
      "didx": pltpu.VMEM((CHUNK,), jnp.int32),
      "rows": pltpu.VMEM((CHUNK, HD), jnp.float32),
      "stage": pltpu.VMEM((RPS, HD), jnp.float32),
      "acc": pltpu.VMEM_SHARED((NPAD, HD), jnp.float32),
      "sem": pltpu.SemaphoreType.DMA,
  }
  if with_counts:
    out_type.append(jax.ShapeDtypeStruct((NPAD, L), jnp.float32))
    scratch.update({
        "ones_v": pltpu.VMEM((CHUNK, L), jnp.float32),
        "cstage": pltpu.VMEM((RPS, L), jnp.float32),
        "cacc": pltpu.VMEM_SHARED((NPAD, L), jnp.float32),
    })

  def body(feats_h, src_h, dst_h, zrow_h, zcnt_h, ones_h, *outs,
           sidx, didx, rows, stage, acc, sem,
           ones_v=None, cstage=None, cacc=None):
    out_h = outs[0]
    core = lax.axis_index("c")
    sub = lax.axis_index("s")
    row0 = sub * RPS

    pltpu.sync_copy(zrow_h, stage)
    pltpu.sync_copy(stage, acc.at[pl.ds(row0, RPS)])
    if with_counts:
      pltpu.sync_copy(ones_h, ones_v)
      pltpu.sync_copy(zcnt_h, cstage)
      pltpu.sync_copy(cstage, cacc.at[pl.ds(row0, RPS)])
    plsc.subcore_barrier()

    def step(k, carry):
      base = sub * EPS + k * CHUNK
      pltpu.sync_copy(src_h.at[pl.ds(base, CHUNK)], sidx)
      pltpu.sync_copy(dst_h.at[pl.ds(base, CHUNK)], didx)
      for g in range(CHUNK // L):
        v = sidx[pl.ds(g * L, L)]
        sidx[pl.ds(g * L, L)] = v * 2 + core
      pltpu.async_copy(feats_h.at[sidx], rows, sem).wait()
      pltpu.sync_copy(rows, acc.at[didx], add=True)
      if with_counts:

        @pl.when(core == 0)
        def _():
          pltpu.sync_copy(ones_v, cacc.at[didx], add=True)

      return carry

    lax.fori_loop(0, STEPS, step, 0)
    plsc.subcore_barrier()

    pltpu.sync_copy(acc.at[pl.ds(row0, RPS)], stage)
    pltpu.sync_copy(stage, out_h.at[pl.ds(core * NPAD + row0, RPS)])
    if with_counts:

      @pl.when(core == 0)
      def _():
        pltpu.sync_copy(cacc.at[pl.ds(row0, RPS)], cstage)
        pltpu.sync_copy(cstage, outs[1].at[pl.ds(row0, RPS)])

  fn = pl.kernel(
      body, out_type=out_type, mesh=mesh, scratch_types=scratch,
      compiler_params=pltpu.CompilerParams(use_tc_tiling_on_sc=False))
  return fn(feats2, src, dst, zrow, zcnt, ones_pat)


def _dense_body(layer1, p_l, p_r, c0, xr, wlt, bl, wrt, scale, beta, o):
  cnt = jnp.maximum(c0[:, 0:1], 1.0)
  agg_l = p_l[...] / cnt
  agg_r = p_r[...] / cnt
  h = (jnp.dot(agg_l, wlt[0:HD, :], preferred_element_type=jnp.float32)
       + jnp.dot(agg_r, wlt[HD:D, :], preferred_element_type=jnp.float32)
       + bl[...]
       + jnp.dot(xr[...], wrt[...], preferred_element_type=jnp.float32))
  nrm = jnp.sqrt(jnp.sum(h * h, axis=1, keepdims=True))
  h = h / jnp.maximum(nrm, 1e-12)
  if layer1:
    h = h * scale[...] + beta[...]
    h = jnp.maximum(h, 0.0)
  o[...] = h


def _dense_call(layer1, P, C, xin, wlt, bl, wrt, scale, beta):
  R = 640
  NB = NPAD // R
  specs = [
      pl.BlockSpec((R, HD), lambda i: (i, 0)),
      pl.BlockSpec((R, HD), lambda i: (i + NB, 0)),
      pl.BlockSpec((R, L), lambda i: (i, 0)),
      pl.BlockSpec((R, D), lambda i: (i, 0)),
      pl.BlockSpec((D, D), lambda i: (0, 0)),
      pl.BlockSpec((1, D), lambda i: (0, 0)),
      pl.BlockSpec((D, D), lambda i: (0, 0)),
      pl.BlockSpec((1, D), lambda i: (0, 0)),
      pl.BlockSpec((1, D), lambda i: (0, 0)),
  ]
  return pl.pallas_call(
      functools.partial(_dense_body, layer1),
      grid=(NB,),
      in_specs=specs,
      out_specs=pl.BlockSpec((R, D), lambda i: (i, 0)),
      out_shape=jax.ShapeDtypeStruct((N, D), jnp.float32),
  )(P, P, C, xin, wlt, bl, wrt, scale, beta)


def kernel(x, edge_index, Wl1, bl1, Wr1, bn_gamma, bn_beta, Wl2, bl2, Wr2):
  src = edge_index[0].astype(jnp.int32)
  dst = edge_index[1].astype(jnp.int32)
  zrow = jnp.zeros((RPS, HD), jnp.float32)
  zcnt = jnp.zeros((RPS, L), jnp.float32)
  ones_pat = jnp.zeros((CHUNK, L), jnp.float32).at[:, 0].set(1.0)

  P1, C = _seg_sum_call(x.reshape(2 * N, HD), src, dst, zrow, zcnt, ones_pat,
                        with_counts=True)
  scale1 = (bn_gamma / jnp.sqrt(1.0 + 1e-5)).reshape(1, D)
  h1 = _dense_call(True, P1, C, x, Wl1.T, bl1.reshape(1, D), Wr1.T,
                   scale1, bn_beta.reshape(1, D))

  (P2,) = _seg_sum_call(h1.reshape(2 * N, HD), src, dst, zrow, zcnt, ones_pat,
                        with_counts=False)
  zb = jnp.zeros((1, D), jnp.float32)
  out = _dense_call(False, P2, C, h1, Wl2.T, bl2.reshape(1, D), Wr2.T, zb, zb)
  return out


# speedup vs baseline: 10.6107x; 3.3053x over previous
import functools

import jax
import jax.numpy as jnp
from jax import lax
from jax.experimental import pallas as pl
from jax.experimental.pallas import tpu as pltpu
from jax.experimental.pallas import tpu_sc as plsc

N = 10000
E = 320000
D = 128
HD = D // 2
NC = 2
NS = 16
L = 16
EPS = E // NS
CHUNK = 80
STEPS = EPS // CHUNK
NBUF = 5
GROUPS = STEPS // NBUF
NPAD = 10240
RPS = NPAD // NS
DRS = 160


def _seg_sum_call(feats2, src4, dst4, zrow, zcnt, ones_pat, with_counts):
  mesh = plsc.VectorSubcoreMesh(
      core_axis_name="c", subcore_axis_name="s", num_cores=NC, num_subcores=NS)

  out_type = [jax.ShapeDtypeStruct((NC * NPAD, HD), jnp.float32)]
  scratch = {
      "sidx": pltpu.VMEM((STEPS, CHUNK), jnp.int32),
      "didx": pltpu.VMEM((STEPS, CHUNK), jnp.int32),
      "stage": pltpu.VMEM((DRS, HD), jnp.float32),
      "acc": pltpu.VMEM_SHARED((NPAD, HD), jnp.float32),
      "isem": pltpu.SemaphoreType.DMA,
  }
  for b in range(NBUF):
    scratch[f"rows{b}"] = pltpu.VMEM((CHUNK, HD), jnp.float32)
    scratch[f"gsem{b}"] = pltpu.SemaphoreType.DMA
    scratch[f"ssem{b}"] = pltpu.SemaphoreType.DMA
    if with_counts:
      scratch[f"csem{b}"] = pltpu.SemaphoreType.DMA
  if with_counts:
    out_type.append(jax.ShapeDtypeStruct((NPAD, L), jnp.float32))
    scratch.update({
        "ones_v": pltpu.VMEM((CHUNK, L), jnp.float32),
        "cstage": pltpu.VMEM((DRS, L), jnp.float32),
        "cacc": pltpu.VMEM_SHARED((NPAD, L), jnp.float32),
    })

  def body(feats_h, src_h, dst_h, zrow_h, zcnt_h, ones_h, *outs, **sc):
    out_h = outs[0]
    core = lax.axis_index("c")
    sub = lax.axis_index("s")
    row0 = sub * RPS
    sidx, didx = sc["sidx"], sc["didx"]
    rows = [sc[f"rows{b}"] for b in range(NBUF)]
    gsem = [sc[f"gsem{b}"] for b in range(NBUF)]
    ssem = [sc[f"ssem{b}"] for b in range(NBUF)]
    acc = sc["acc"]

    pltpu.sync_copy(src_h.at[pl.ds(sub * STEPS, STEPS)], sidx)
    pltpu.sync_copy(dst_h.at[pl.ds(sub * STEPS, STEPS)], didx)

    def remap(r, carry):
      for g in range(CHUNK // L):
        v = sidx[r, pl.ds(g * L, L)]
        sidx[r, pl.ds(g * L, L)] = v * 2 + core
      return carry

    lax.fori_loop(0, STEPS, remap, 0)

    for j in range(RPS // DRS):
      pltpu.sync_copy(zrow_h, sc["stage"])
      pltpu.sync_copy(sc["stage"], acc.at[pl.ds(row0 + j * DRS, DRS)])
    if with_counts:
      pltpu.sync_copy(ones_h, sc["ones_v"])
      for j in range(RPS // DRS):
        pltpu.sync_copy(zcnt_h, sc["cstage"])
        pltpu.sync_copy(sc["cstage"], sc["cacc"].at[pl.ds(row0 + j * DRS, DRS)])
    plsc.subcore_barrier()

    for b in range(NBUF):
      pltpu.async_copy(feats_h.at[sidx.at[b]], rows[b], gsem[b])

    def group(j, carry):
      k0 = j * NBUF
      handles = []
      for b in range(NBUF):
        k = k0 + b
        pltpu.make_async_copy(feats_h.at[sidx.at[k]], rows[b], gsem[b]).wait()
        handles.append(
            pltpu.async_copy(rows[b], acc.at[didx.at[k]], sem=ssem[b],
                             add=True))
        if with_counts:

          @pl.when(core == 0)
          def _():
            pltpu.async_copy(sc["ones_v"], sc["cacc"].at[didx.at[k]],
                             sem=sc[f"csem{b}"], add=True)

      for b in range(NBUF):
        k = k0 + b
        handles[b].wait()
        if with_counts:

          @pl.when(core == 0)
          def _():
            pltpu.make_async_copy(
                sc["ones_v"], sc["cacc"].at[didx.at[k]], sc[f"csem{b}"]).wait()

        @pl.when(k + NBUF < STEPS)
        def _():
          pltpu.async_copy(feats_h.at[sidx.at[k + NBUF]], rows[b], gsem[b])

      return carry

    lax.fori_loop(0, GROUPS, group, 0)
    plsc.subcore_barrier()

    for j in range(RPS // DRS):
      pltpu.sync_copy(acc.at[pl.ds(row0 + j * DRS, DRS)], sc["stage"])
      pltpu.sync_copy(sc["stage"],
                      out_h.at[pl.ds(core * NPAD + row0 + j * DRS, DRS)])
    if with_counts:

      @pl.when(core == 0)
      def _():
        for j in range(RPS // DRS):
          pltpu.sync_copy(sc["cacc"].at[pl.ds(row0 + j * DRS, DRS)],
                          sc["cstage"])
          pltpu.sync_copy(sc["cstage"],
                          outs[1].at[pl.ds(row0 + j * DRS, DRS)])

  fn = pl.kernel(
      body, out_type=out_type, mesh=mesh, scratch_types=scratch,
      compiler_params=pltpu.CompilerParams(use_tc_tiling_on_sc=False))
  return fn(feats2, src4, dst4, zrow, zcnt, ones_pat)


def _dense_body(layer1, p_l, p_r, c0, xr, wlt, bl, wrt, scale, beta, o):
  cnt = jnp.maximum(c0[:, 0:1], 1.0)
  agg_l = p_l[...] / cnt
  agg_r = p_r[...] / cnt
  h = (jnp.dot(agg_l, wlt[0:HD, :], preferred_element_type=jnp.float32)
       + jnp.dot(agg_r, wlt[HD:D, :], preferred_element_type=jnp.float32)
       + bl[...]
       + jnp.dot(xr[...], wrt[...], preferred_element_type=jnp.float32))
  nrm = jnp.sqrt(jnp.sum(h * h, axis=1, keepdims=True))
  h = h / jnp.maximum(nrm, 1e-12)
  if layer1:
    h = h * scale[...] + beta[...]
    h = jnp.maximum(h, 0.0)
  o[...] = h


def _dense_call(layer1, P, C, xin, wlt, bl, wrt, scale, beta):
  R = 640
  NB = NPAD // R
  specs = [
      pl.BlockSpec((R, HD), lambda i: (i, 0)),
      pl.BlockSpec((R, HD), lambda i: (i + NB, 0)),
      pl.BlockSpec((R, L), lambda i: (i, 0)),
      pl.BlockSpec((R, D), lambda i: (i, 0)),
      pl.BlockSpec((D, D), lambda i: (0, 0)),
      pl.BlockSpec((1, D), lambda i: (0, 0)),
      pl.BlockSpec((D, D), lambda i: (0, 0)),
      pl.BlockSpec((1, D), lambda i: (0, 0)),
      pl.BlockSpec((1, D), lambda i: (0, 0)),
  ]
  return pl.pallas_call(
      functools.partial(_dense_body, layer1),
      grid=(NB,),
      in_specs=specs,
      out_specs=pl.BlockSpec((R, D), lambda i: (i, 0)),
      out_shape=jax.ShapeDtypeStruct((N, D), jnp.float32),
  )(P, P, C, xin, wlt, bl, wrt, scale, beta)


def kernel(x, edge_index, Wl1, bl1, Wr1, bn_gamma, bn_beta, Wl2, bl2, Wr2):
  src4 = edge_index[0].astype(jnp.int32).reshape(NS * STEPS, CHUNK)
  dst4 = edge_index[1].astype(jnp.int32).reshape(NS * STEPS, CHUNK)
  zrow = jnp.zeros((DRS, HD), jnp.float32)
  zcnt = jnp.zeros((DRS, L), jnp.float32)
  ones_pat = jnp.zeros((CHUNK, L), jnp.float32).at[:, 0].set(1.0)

  P1, C = _seg_sum_call(x.reshape(2 * N, HD), src4, dst4, zrow, zcnt,
                        ones_pat, with_counts=True)
  scale1 = (bn_gamma / jnp.sqrt(1.0 + 1e-5)).reshape(1, D)
  h1 = _dense_call(True, P1, C, x, Wl1.T, bl1.reshape(1, D), Wr1.T,
                   scale1, bn_beta.reshape(1, D))

  (P2,) = _seg_sum_call(h1.reshape(2 * N, HD), src4, dst4, zrow, zcnt,
                        ones_pat, with_counts=False)
  zb = jnp.zeros((1, D), jnp.float32)
  out = _dense_call(False, P2, C, h1, Wl2.T, bl2.reshape(1, D), Wr2.T, zb, zb)
  return out
